# baseline (speedup 1.0000x reference)
import jax
import jax.numpy as jnp
from jax.experimental import pallas as pl
from jax.experimental.pallas import tpu as pltpu

B = 32
N = 32
HS = 501
NZ = 56
HSP = 512
NZP = 128


def _pad2(a, r, c):
    return jnp.pad(a, ((0, r - a.shape[0]), (0, c - a.shape[1])))


def _pad1(a, n):
    return jnp.pad(a, (0, n - a.shape[0]))


def _dvae_body(X_ref, adj_ref, wi3_ref, bi3_ref, whh_ref, bh3_ref,
               wgm_ref, gme_ref, bgm_ref, wf_ref, bf_ref, out_ref, G):
    G[...] = jnp.zeros_like(G)
    j_iota = jax.lax.broadcasted_iota(jnp.int32, (B, N * B), 1)
    b_iota = jax.lax.broadcasted_iota(jnp.int32, (B, N * B), 0)
    eye_mask = (j_iota % B == b_iota)
    u_of_j = j_iota // B

    def step(v, _):
        M = jnp.where(eye_mask & (u_of_j < v), adj_ref[pl.ds(v, 1)][0], 0.0)
        Hagg = jnp.dot(M, G[...], preferred_element_type=jnp.float32)
        xv = X_ref[pl.ds(v, 1)][0][:, 0:1]
        gi = xv * wi3_ref[...] + bi3_ref[...]
        gh = jnp.dot(Hagg, whh_ref[...],
                     preferred_element_type=jnp.float32) + bh3_ref[...]
        r = jax.nn.sigmoid(gi[:, :HSP] + gh[:, :HSP])
        z = jax.nn.sigmoid(gi[:, HSP:2 * HSP] + gh[:, HSP:2 * HSP])
        n = jnp.tanh(gi[:, 2 * HSP:] + r * gh[:, 2 * HSP:])
        Hv = (1.0 - z) * n + z * Hagg
        gm = (jnp.dot(Hv, wgm_ref[...], preferred_element_type=jnp.float32)
              + gme_ref[pl.ds(v, 1)] + bgm_ref[...])
        G[pl.ds(v * B, B)] = jax.nn.sigmoid(gm[:, :HSP]) * gm[:, HSP:]
        return Hv

    Hlast = jax.lax.fori_loop(0, N, step, jnp.zeros((B, HSP), jnp.float32))
    out_ref[...] = jnp.dot(Hlast, wf_ref[...],
                           preferred_element_type=jnp.float32) + bf_ref[...]


def kernel(x, adj, W_ih, W_hh, b_ih, b_hh, Wg, bg, Wm, Wf, bf):
    f32 = jnp.float32
    wih = W_ih[:, 0]
    wi3 = jnp.concatenate(
        [_pad1(wih[k * HS:(k + 1) * HS], HSP) for k in range(3)])[None]
    bi3 = jnp.concatenate(
        [_pad1(b_ih[k * HS:(k + 1) * HS], HSP) for k in range(3)])[None]
    bh3 = jnp.concatenate(
        [_pad1(b_hh[k * HS:(k + 1) * HS], HSP) for k in range(3)])[None]
    whh = jnp.concatenate(
        [_pad2(W_hh[k * HS:(k + 1) * HS].T, HSP, HSP) for k in range(3)],
        axis=1)
    wgm = jnp.concatenate(
        [_pad2(Wg[:, :HS].T, HSP, HSP), _pad2(Wm[:, :HS].T, HSP, HSP)],
        axis=1)
    gme = jnp.concatenate(
        [_pad2(Wg[:, HS:].T, N, HSP), _pad2(Wm[:, HS:].T, N, HSP)],
        axis=1)
    bgm = jnp.concatenate(
        [_pad1(bg, HSP), jnp.zeros((HSP,), f32)])[None]
    wf = _pad2(Wf.T, HSP, NZP)
    bfp = _pad1(bf, NZP)[None]
    X = jnp.broadcast_to(x.T[:, :, None], (N, B, 128)).astype(f32)
    adjf = jnp.transpose(adj, (2, 1, 0)).astype(f32).reshape(N, 1, N * B)

    out = pl.pallas_call(
        _dvae_body,
        out_shape=jax.ShapeDtypeStruct((B, NZP), f32),
        scratch_shapes=[pltpu.VMEM((N * B, HSP), f32)],
    )(X, adjf, wi3, bi3, whh, bh3, wgm, gme, bgm, wf, bfp)
    return out[:, :NZ][:, :, None]


# speedup vs baseline: 1.1126x; 1.1126x over previous
import jax
import jax.numpy as jnp
from jax.experimental import pallas as pl
from jax.experimental.pallas import tpu as pltpu

B = 32
N = 32
HS = 501
NZ = 56
HSP = 512
NZP = 128


def _pad2(a, r, c):
    return jnp.pad(a, ((0, r - a.shape[0]), (0, c - a.shape[1])))


def _pad1(a, n):
    return jnp.pad(a, (0, n - a.shape[0]))


def _dvae_body(X_ref, dc_ref, ash_ref, wi3_ref, bi3_ref, whh_ref, bh3_ref,
               wgm_ref, gme_ref, bgm_ref, wf_ref, bf_ref, out_ref, G):
    G[...] = jnp.zeros_like(G)
    s_iota = jax.lax.broadcasted_iota(jnp.int32, (N, B), 0)

    def step(w, Hprev):
        coef = jnp.where(s_iota < w, ash_ref[pl.ds(w, 1)][0], 0.0)
        stable = jnp.sum(coef[:, :, None] * G[...], axis=0)
        gm = (jnp.dot(Hprev, wgm_ref[...], preferred_element_type=jnp.float32)
              + gme_ref[pl.ds(w, 1)] + bgm_ref[...])
        gated = jax.nn.sigmoid(gm[:, :HSP]) * gm[:, HSP:]
        G[pl.ds(w, 1)] = gated[None]
        cc = dc_ref[pl.ds(w, 1)][0][:, 0:1]
        Hagg = stable + cc * gated
        xv = X_ref[pl.ds(w, 1)][0][:, 0:1]
        gi = xv * wi3_ref[...] + bi3_ref[...]
        gh = jnp.dot(Hagg, whh_ref[...],
                     preferred_element_type=jnp.float32) + bh3_ref[...]
        r = jax.nn.sigmoid(gi[:, :HSP] + gh[:, :HSP])
        z = jax.nn.sigmoid(gi[:, HSP:2 * HSP] + gh[:, HSP:2 * HSP])
        n = jnp.tanh(gi[:, 2 * HSP:] + r * gh[:, 2 * HSP:])
        return (1.0 - z) * n + z * Hagg

    Hlast = jax.lax.fori_loop(0, N, step, jnp.zeros((B, HSP), jnp.float32))
    out_ref[...] = jnp.dot(Hlast, wf_ref[...],
                           preferred_element_type=jnp.float32) + bf_ref[...]


def kernel(x, adj, W_ih, W_hh, b_ih, b_hh, Wg, bg, Wm, Wf, bf):
    f32 = jnp.float32
    wih = W_ih[:, 0]
    wi3 = jnp.concatenate(
        [_pad1(wih[k * HS:(k + 1) * HS], HSP) for k in range(3)])[None]
    bi3 = jnp.concatenate(
        [_pad1(b_ih[k * HS:(k + 1) * HS], HSP) for k in range(3)])[None]
    bh3 = jnp.concatenate(
        [_pad1(b_hh[k * HS:(k + 1) * HS], HSP) for k in range(3)])[None]
    whh = jnp.concatenate(
        [_pad2(W_hh[k * HS:(k + 1) * HS].T, HSP, HSP) for k in range(3)],
        axis=1)
    wgm = jnp.concatenate(
        [_pad2(Wg[:, :HS].T, HSP, HSP), _pad2(Wm[:, :HS].T, HSP, HSP)],
        axis=1)
    gme = jnp.concatenate(
        [_pad2(Wg[:, HS:].T, N, HSP), _pad2(Wm[:, HS:].T, N, HSP)],
        axis=1)
    gme_sh = jnp.concatenate([jnp.zeros((1, 2 * HSP), f32), gme[:N - 1]])
    bgm = jnp.concatenate(
        [_pad1(bg, HSP), jnp.zeros((HSP,), f32)])[None]
    wf = _pad2(Wf.T, HSP, NZP)
    bfp = _pad1(bf, NZP)[None]
    adj_eff = adj.astype(f32) * jnp.triu(jnp.ones((N, N), f32), k=1)
    a_t = jnp.transpose(adj_eff, (2, 1, 0))
    ash = jnp.concatenate([jnp.zeros((N, 1, B), f32), a_t[:, :N - 1, :]], axis=1)
    dc = jnp.concatenate(
        [jnp.zeros((B, 1), f32),
         jnp.diagonal(adj_eff, offset=1, axis1=1, axis2=2)], axis=1)
    dcb = jnp.broadcast_to(dc.T[:, :, None], (N, B, 128))
    X = jnp.broadcast_to(x.T[:, :, None], (N, B, 128)).astype(f32)

    out = pl.pallas_call(
        _dvae_body,
        out_shape=jax.ShapeDtypeStruct((B, NZP), f32),
        scratch_shapes=[pltpu.VMEM((N, B, HSP), f32)],
    )(X, dcb, ash, wi3, bi3, whh, bh3, wgm, gme_sh, bgm, wf, bfp)
    return out[:, :NZ][:, :, None]
